# baseline (speedup 1.0000x reference)
import functools

import jax
import jax.numpy as jnp
from jax import lax
from jax.experimental import pallas as pl
from jax.experimental.pallas import tpu as pltpu
from jax.experimental.pallas import tpu_sc as plsc

N, E, D = 10000, 320000, 128
E2 = 327680
N2 = 10240
NC, NS = 2, 16
NW = NC * NS
EPW = E2 // NW
CH = 80
NCHUNK = EPW // CH
RPT = N2 // NS
BLK = 1024
NBLK = N2 // BLK

_MESH = plsc.VectorSubcoreMesh(core_axis_name="c", subcore_axis_name="s")


@functools.partial(
    pl.kernel,
    out_type=jax.ShapeDtypeStruct((NC * N2,), jnp.float32),
    mesh=_MESH,
    scratch_types=[
        pltpu.VMEM((CH,), jnp.int32),
        pltpu.VMEM((CH,), jnp.int32),
        pltpu.VMEM((CH,), jnp.float32),
        pltpu.VMEM_SHARED((N2,), jnp.float32),
    ] + [pltpu.SemaphoreType.DMA] * 4,
)
def _sc_hist(dst_hbm, zeros_hbm, deg_hbm, idx0, idx1, ones_v, acc,
             isem0, isem1, ssem0, ssem1):
    c = lax.axis_index("c")
    s = lax.axis_index("s")
    wid = c * NS + s
    r0 = s * RPT
    pltpu.sync_copy(zeros_hbm.at[pl.ds(r0, RPT)], acc.at[pl.ds(r0, RPT)])
    for k in range(CH // 16):
        ones_v[pl.ds(k * 16, 16)] = jnp.full((16,), 1.0, jnp.float32)
    plsc.subcore_barrier()

    idx = (idx0, idx1)
    isem = (isem0, isem1)
    ssem = (ssem0, ssem1)
    base = wid * EPW

    def eoff(i):
        return pl.multiple_of(base + i * CH, 8)

    pltpu.sync_copy(dst_hbm.at[pl.ds(eoff(0), CH)], idx0)

    def step(i, p):
        @pl.when(i >= 1)
        def _():
            pltpu.make_async_copy(dst_hbm.at[pl.ds(eoff(i), CH)],
                                  idx[p], isem[p]).wait()
        pltpu.async_copy(ones_v, acc.at[idx[p]], ssem[p], add=True)

        @pl.when(i >= 1)
        def _():
            pltpu.make_async_copy(ones_v, acc.at[idx[1 - p]],
                                  ssem[1 - p]).wait()

        @pl.when(i + 1 < NCHUNK)
        def _():
            pltpu.async_copy(dst_hbm.at[pl.ds(eoff(i + 1), CH)],
                             idx[1 - p], isem[1 - p])

    def body(j, carry):
        step(2 * j, 0)
        step(2 * j + 1, 1)
        return carry

    lax.fori_loop(0, NCHUNK // 2, body, 0)
    pltpu.make_async_copy(ones_v, acc.at[idx[(NCHUNK - 1) % 2]],
                          ssem[(NCHUNK - 1) % 2]).wait()
    plsc.subcore_barrier()
    pltpu.sync_copy(acc.at[pl.ds(r0, RPT)],
                    deg_hbm.at[pl.ds(c * N2 + r0, RPT)])


@functools.partial(
    pl.kernel,
    out_type=jax.ShapeDtypeStruct((NC * N2, D), jnp.float32),
    mesh=_MESH,
    scratch_types=[pltpu.VMEM((CH,), jnp.int32)] * 4
    + [pltpu.VMEM((CH,), jnp.int32)] * 4
    + [pltpu.VMEM((CH, D), jnp.float32)] * 4
    + [pltpu.VMEM_SHARED((N2, D), jnp.float32)]
    + [pltpu.SemaphoreType.DMA] * 16,
)
def _sc_msg(g_hbm, srcf_hbm, dstf_hbm, zeros_hbm, q_hbm,
            idxs0, idxs1, idxs2, idxs3, idxd0, idxd1, idxd2, idxd3,
            rows0, rows1, rows2, rows3, acc,
            gsem0, gsem1, gsem2, gsem3, isem0, isem1, isem2, isem3,
            dsem0, dsem1, dsem2, dsem3, ssem0, ssem1, ssem2, ssem3):
    c = lax.axis_index("c")
    s = lax.axis_index("s")
    wid = c * NS + s
    r0 = s * RPT
    pltpu.sync_copy(zeros_hbm.at[pl.ds(r0, RPT)], acc.at[pl.ds(r0, RPT)])
    plsc.subcore_barrier()

    idxs = (idxs0, idxs1, idxs2, idxs3)
    idxd = (idxd0, idxd1, idxd2, idxd3)
    rows = (rows0, rows1, rows2, rows3)
    gsem = (gsem0, gsem1, gsem2, gsem3)
    isem = (isem0, isem1, isem2, isem3)
    dsem = (dsem0, dsem1, dsem2, dsem3)
    ssem = (ssem0, ssem1, ssem2, ssem3)
    base = wid * EPW

    def eoff(i):
        return pl.multiple_of(base + i * CH, 8)

    def sidx_load(i, p, sem_tuple=None):
        pltpu.async_copy(srcf_hbm.at[pl.ds(eoff(i), CH)], idxs[p], isem[p])

    def sidx_wait(i, p):
        pltpu.make_async_copy(srcf_hbm.at[pl.ds(eoff(i), CH)],
                              idxs[p], isem[p]).wait()

    def didx_load(i, p):
        pltpu.async_copy(dstf_hbm.at[pl.ds(eoff(i), CH)], idxd[p], dsem[p])

    def didx_wait(i, p):
        pltpu.make_async_copy(dstf_hbm.at[pl.ds(eoff(i), CH)],
                              idxd[p], dsem[p]).wait()

    def gather(i, p):
        pltpu.async_copy(g_hbm.at[idxs[p]], rows[p], gsem[p])

    def gather_wait(i, p):
        pltpu.make_async_copy(g_hbm.at[idxs[p]], rows[p], gsem[p]).wait()

    def scatter(i, p):
        pltpu.async_copy(rows[p], acc.at[idxd[p]], ssem[p], add=True)

    def scatter_wait(i, p):
        pltpu.make_async_copy(rows[p], acc.at[idxd[p]], ssem[p]).wait()

    pltpu.sync_copy(srcf_hbm.at[pl.ds(eoff(0), CH)], idxs0)
    pltpu.sync_copy(srcf_hbm.at[pl.ds(eoff(1), CH)], idxs1)
    sidx_load(2, 2)
    sidx_load(3, 3)
    didx_load(0, 0)
    didx_load(1, 1)
    gather(0, 0)
    gather(1, 1)

    def step(i, p):
        q = (p + 2) % 4
        gather_wait(i, p)

        @pl.when(i + 4 < NCHUNK)
        def _():
            sidx_load(i + 4, p)

        didx_wait(i, p)
        scatter(i, p)

        @pl.when(i >= 2)
        def _():
            scatter_wait(i - 2, q)

        @pl.when(i + 2 < NCHUNK)
        def _():
            sidx_wait(i + 2, q)
            gather(i + 2, q)
            didx_load(i + 2, q)

    def body(j, carry):
        for p in range(4):
            step(4 * j + p, p)
        return carry

    lax.fori_loop(0, NCHUNK // 4, body, 0)
    scatter_wait(NCHUNK - 2, (NCHUNK - 2) % 4)
    scatter_wait(NCHUNK - 1, (NCHUNK - 1) % 4)
    plsc.subcore_barrier()
    pltpu.sync_copy(acc.at[pl.ds(r0, RPT)],
                    q_hbm.at[pl.ds(c * N2 + r0, RPT)])


def _tc_scale_body(x_ref, w_ref, degp_ref, g_ref, dinv_ref):
    deg = 1.0 + degp_ref[0] + degp_ref[1]
    di = lax.rsqrt(deg)
    h = jnp.dot(x_ref[...], w_ref[...], preferred_element_type=jnp.float32)
    g_ref[...] = h * di
    dinv_ref[...] = di


def _tc_scale(x2, W, degp):
    return pl.pallas_call(
        _tc_scale_body,
        grid=(NBLK,),
        in_specs=[
            pl.BlockSpec((BLK, D), lambda i: (i, 0)),
            pl.BlockSpec((D, D), lambda i: (0, 0)),
            pl.BlockSpec((NC, BLK, 1), lambda i: (0, i, 0)),
        ],
        out_specs=[
            pl.BlockSpec((BLK, D), lambda i: (i, 0)),
            pl.BlockSpec((BLK, 1), lambda i: (i, 0)),
        ],
        out_shape=[
            jax.ShapeDtypeStruct((N2, D), jnp.float32),
            jax.ShapeDtypeStruct((N2, 1), jnp.float32),
        ],
    )(x2, W, degp)


def _tc_combine_body(q0_ref, q1_ref, g_ref, dinv_ref, b_ref, o_ref):
    acc = q0_ref[...] + q1_ref[...] + g_ref[...]
    o_ref[...] = jnp.maximum(acc * dinv_ref[...] + b_ref[...], 0.0)


def _tc_combine(qf, g, dinv, b2):
    return pl.pallas_call(
        _tc_combine_body,
        grid=(NBLK,),
        in_specs=[
            pl.BlockSpec((BLK, D), lambda i: (i, 0)),
            pl.BlockSpec((BLK, D), lambda i: (i + NBLK, 0)),
            pl.BlockSpec((BLK, D), lambda i: (i, 0)),
            pl.BlockSpec((BLK, 1), lambda i: (i, 0)),
            pl.BlockSpec((1, D), lambda i: (0, 0)),
        ],
        out_specs=pl.BlockSpec((BLK, D), lambda i: (i, 0)),
        out_shape=jax.ShapeDtypeStruct((N2, D), jnp.float32),
    )(qf, qf, g, dinv, b2)


def kernel(x, edge_index, W, b):
    pad_idx = N + jnp.arange(E2 - E, dtype=jnp.int32) % (N2 - N)
    ei2 = jnp.concatenate(
        [edge_index, jnp.stack([pad_idx, pad_idx])], axis=1)
    src = ei2[0]
    dst = ei2[1]
    x2 = jnp.pad(x, ((0, N2 - N), (0, 0)))
    z1 = jnp.zeros((N2,), jnp.float32)
    zD = jnp.zeros((N2, D), jnp.float32)

    degp = _sc_hist(dst, z1).reshape(NC, N2, 1)
    g, dinv = _tc_scale(x2, W, degp)
    qf = _sc_msg(g, src, dst, zD)
    out = _tc_combine(qf, g, dinv, b.reshape(1, D))
    return out[:N]


# speedup vs baseline: 1.0132x; 1.0132x over previous
import functools

import jax
import jax.numpy as jnp
from jax import lax
from jax.experimental import pallas as pl
from jax.experimental.pallas import tpu as pltpu
from jax.experimental.pallas import tpu_sc as plsc

N, E, D = 10000, 320000, 128
E2 = 327680
N2 = 10240
NC, NS = 2, 16
NW = NC * NS
EPW = E2 // NW
CH = 128
NCHUNK = EPW // CH
RPT = N2 // NS
BLK = 1024
NBLK = N2 // BLK

_MESH = plsc.VectorSubcoreMesh(core_axis_name="c", subcore_axis_name="s")


@functools.partial(
    pl.kernel,
    out_type=jax.ShapeDtypeStruct((NC * N2,), jnp.float32),
    mesh=_MESH,
    scratch_types=[
        pltpu.VMEM((CH,), jnp.int32),
        pltpu.VMEM((CH,), jnp.int32),
        pltpu.VMEM((CH,), jnp.float32),
        pltpu.VMEM_SHARED((N2,), jnp.float32),
    ] + [pltpu.SemaphoreType.DMA] * 4,
)
def _sc_hist(dst_hbm, zeros_hbm, deg_hbm, idx0, idx1, ones_v, acc,
             isem0, isem1, ssem0, ssem1):
    c = lax.axis_index("c")
    s = lax.axis_index("s")
    wid = c * NS + s
    r0 = s * RPT
    pltpu.sync_copy(zeros_hbm.at[pl.ds(r0, RPT)], acc.at[pl.ds(r0, RPT)])
    for k in range(CH // 16):
        ones_v[pl.ds(k * 16, 16)] = jnp.full((16,), 1.0, jnp.float32)
    plsc.subcore_barrier()

    idx = (idx0, idx1)
    isem = (isem0, isem1)
    ssem = (ssem0, ssem1)
    base = wid * EPW

    def eoff(i):
        return pl.multiple_of(base + i * CH, 8)

    pltpu.sync_copy(dst_hbm.at[pl.ds(eoff(0), CH)], idx0)

    def step(i, p):
        @pl.when(i >= 1)
        def _():
            pltpu.make_async_copy(dst_hbm.at[pl.ds(eoff(i), CH)],
                                  idx[p], isem[p]).wait()
        pltpu.async_copy(ones_v, acc.at[idx[p]], ssem[p], add=True)

        @pl.when(i >= 1)
        def _():
            pltpu.make_async_copy(ones_v, acc.at[idx[1 - p]],
                                  ssem[1 - p]).wait()

        @pl.when(i + 1 < NCHUNK)
        def _():
            pltpu.async_copy(dst_hbm.at[pl.ds(eoff(i + 1), CH)],
                             idx[1 - p], isem[1 - p])

    def body(j, carry):
        step(2 * j, 0)
        step(2 * j + 1, 1)
        return carry

    lax.fori_loop(0, NCHUNK // 2, body, 0)
    pltpu.make_async_copy(ones_v, acc.at[idx[(NCHUNK - 1) % 2]],
                          ssem[(NCHUNK - 1) % 2]).wait()
    plsc.subcore_barrier()
    pltpu.sync_copy(acc.at[pl.ds(r0, RPT)],
                    deg_hbm.at[pl.ds(c * N2 + r0, RPT)])


@functools.partial(
    pl.kernel,
    out_type=jax.ShapeDtypeStruct((NC * N2, D), jnp.float32),
    mesh=_MESH,
    scratch_types=[
        pltpu.VMEM((CH,), jnp.int32),
        pltpu.VMEM((CH,), jnp.int32),
        pltpu.VMEM((CH,), jnp.int32),
        pltpu.VMEM((CH,), jnp.int32),
        pltpu.VMEM((CH, D), jnp.float32),
        pltpu.VMEM((CH, D), jnp.float32),
        pltpu.VMEM_SHARED((N2, D), jnp.float32),
    ] + [pltpu.SemaphoreType.DMA] * 6,
)
def _sc_msg(g_hbm, srcf_hbm, dstf_hbm, zeros_hbm, q_hbm,
            idxs0, idxs1, idxd0, idxd1, rows0, rows1, acc,
            gsem0, gsem1, isem0, isem1, dsem0, dsem1):
    c = lax.axis_index("c")
    s = lax.axis_index("s")
    wid = c * NS + s
    r0 = s * RPT
    pltpu.sync_copy(zeros_hbm.at[pl.ds(r0, RPT)], acc.at[pl.ds(r0, RPT)])
    plsc.subcore_barrier()

    rows = (rows0, rows1)
    idxs = (idxs0, idxs1)
    idxd = (idxd0, idxd1)
    gsem = (gsem0, gsem1)
    isem = (isem0, isem1)
    dsem = (dsem0, dsem1)
    base = wid * EPW

    def eoff(i):
        return pl.multiple_of(base + i * CH, 8)

    pltpu.sync_copy(srcf_hbm.at[pl.ds(eoff(0), CH)], idxs0)
    pltpu.async_copy(dstf_hbm.at[pl.ds(eoff(0), CH)], idxd0, dsem0)
    pltpu.async_copy(g_hbm.at[idxs0], rows0, gsem0)
    pltpu.async_copy(srcf_hbm.at[pl.ds(eoff(1), CH)], idxs1, isem1)

    def step(i, p):
        pltpu.make_async_copy(g_hbm.at[idxs[p]], rows[p], gsem[p]).wait()

        @pl.when(i + 2 < NCHUNK)
        def _():
            pltpu.async_copy(srcf_hbm.at[pl.ds(eoff(i + 2), CH)],
                             idxs[p], isem[p])

        @pl.when(i + 1 < NCHUNK)
        def _():
            pltpu.make_async_copy(srcf_hbm.at[pl.ds(eoff(i + 1), CH)],
                                  idxs[1 - p], isem[1 - p]).wait()
            pltpu.async_copy(g_hbm.at[idxs[1 - p]], rows[1 - p], gsem[1 - p])
            pltpu.async_copy(dstf_hbm.at[pl.ds(eoff(i + 1), CH)],
                             idxd[1 - p], dsem[1 - p])

        pltpu.make_async_copy(dstf_hbm.at[pl.ds(eoff(i), CH)],
                              idxd[p], dsem[p]).wait()
        pltpu.sync_copy(rows[p], acc.at[idxd[p]], add=True)

    def body(j, carry):
        step(2 * j, 0)
        step(2 * j + 1, 1)
        return carry

    lax.fori_loop(0, NCHUNK // 2, body, 0)
    plsc.subcore_barrier()
    pltpu.sync_copy(acc.at[pl.ds(r0, RPT)],
                    q_hbm.at[pl.ds(c * N2 + r0, RPT)])


def _tc_matmul_body(x_ref, w_ref, h_ref):
    h_ref[...] = jnp.dot(x_ref[...], w_ref[...],
                         preferred_element_type=jnp.float32)


def _tc_matmul(x2, W):
    return pl.pallas_call(
        _tc_matmul_body,
        grid=(NBLK,),
        in_specs=[
            pl.BlockSpec((BLK, D), lambda i: (i, 0)),
            pl.BlockSpec((D, D), lambda i: (0, 0)),
        ],
        out_specs=pl.BlockSpec((BLK, D), lambda i: (i, 0)),
        out_shape=jax.ShapeDtypeStruct((N2, D), jnp.float32),
    )(x2, W)


def _tc_scale_body(h_ref, degp_ref, g_ref, dinv_ref):
    deg = 1.0 + degp_ref[0] + degp_ref[1]
    di = lax.rsqrt(deg)
    g_ref[...] = h_ref[...] * di
    dinv_ref[...] = di


def _tc_scale(h, degp):
    return pl.pallas_call(
        _tc_scale_body,
        grid=(NBLK,),
        in_specs=[
            pl.BlockSpec((BLK, D), lambda i: (i, 0)),
            pl.BlockSpec((NC, BLK, 1), lambda i: (0, i, 0)),
        ],
        out_specs=[
            pl.BlockSpec((BLK, D), lambda i: (i, 0)),
            pl.BlockSpec((BLK, 1), lambda i: (i, 0)),
        ],
        out_shape=[
            jax.ShapeDtypeStruct((N2, D), jnp.float32),
            jax.ShapeDtypeStruct((N2, 1), jnp.float32),
        ],
    )(h, degp)


def _tc_combine_body(q0_ref, q1_ref, g_ref, dinv_ref, b_ref, o_ref):
    acc = q0_ref[...] + q1_ref[...] + g_ref[...]
    o_ref[...] = jnp.maximum(acc * dinv_ref[...] + b_ref[...], 0.0)


def _tc_combine(qf, g, dinv, b2):
    return pl.pallas_call(
        _tc_combine_body,
        grid=(NBLK,),
        in_specs=[
            pl.BlockSpec((BLK, D), lambda i: (i, 0)),
            pl.BlockSpec((BLK, D), lambda i: (i + NBLK, 0)),
            pl.BlockSpec((BLK, D), lambda i: (i, 0)),
            pl.BlockSpec((BLK, 1), lambda i: (i, 0)),
            pl.BlockSpec((1, D), lambda i: (0, 0)),
        ],
        out_specs=pl.BlockSpec((BLK, D), lambda i: (i, 0)),
        out_shape=jax.ShapeDtypeStruct((N2, D), jnp.float32),
    )(qf, qf, g, dinv, b2)


def kernel(x, edge_index, W, b):
    pad_idx = N + jnp.arange(E2 - E, dtype=jnp.int32) % (N2 - N)
    ei2 = jnp.concatenate(
        [edge_index, jnp.stack([pad_idx, pad_idx])], axis=1)
    src = ei2[0]
    dst = ei2[1]
    x2 = jnp.pad(x, ((0, N2 - N), (0, 0)))
    z1 = jnp.zeros((N2,), jnp.float32)
    zD = jnp.zeros((N2, D), jnp.float32)

    h = _tc_matmul(x2, W)
    degp = _sc_hist(dst, z1).reshape(NC, N2, 1)
    g, dinv = _tc_scale(h, degp)
    qf = _sc_msg(g, src, dst, zD)
    out = _tc_combine(qf, g, dinv, b.reshape(1, D))
    return out[:N]


# speedup vs baseline: 1.1526x; 1.1376x over previous
import functools

import jax
import jax.numpy as jnp
from jax import lax
from jax.experimental import pallas as pl
from jax.experimental.pallas import tpu as pltpu
from jax.experimental.pallas import tpu_sc as plsc

N, E, D = 10000, 320000, 128
E2 = 327680
N2 = 10240
NC, NS = 2, 16
NW = NC * NS
EPW = E2 // NW
CH = 128
NCHUNK = EPW // CH
RPT = N2 // NS
BLK = 1024
NBLK = N2 // BLK

_MESH = plsc.VectorSubcoreMesh(core_axis_name="c", subcore_axis_name="s")


_NHB = 8


@functools.partial(
    pl.kernel,
    out_type=jax.ShapeDtypeStruct((NC * N2,), jnp.float32),
    mesh=_MESH,
    scratch_types=[pltpu.VMEM((CH,), jnp.int32)] * _NHB
    + [
        pltpu.VMEM((CH,), jnp.float32),
        pltpu.VMEM_SHARED((N2,), jnp.float32),
    ]
    + [pltpu.SemaphoreType.DMA] * (2 * _NHB),
)
def _sc_hist(dst_hbm, zeros_hbm, deg_hbm, *refs):
    idx = refs[:_NHB]
    ones_v = refs[_NHB]
    acc = refs[_NHB + 1]
    isem = refs[_NHB + 2:_NHB + 2 + _NHB]
    ssem = refs[_NHB + 2 + _NHB:]
    c = lax.axis_index("c")
    s = lax.axis_index("s")
    wid = c * NS + s
    r0 = s * RPT
    pltpu.sync_copy(zeros_hbm.at[pl.ds(r0, RPT)], acc.at[pl.ds(r0, RPT)])
    for k in range(CH // 16):
        ones_v[pl.ds(k * 16, 16)] = jnp.full((16,), 1.0, jnp.float32)
    plsc.subcore_barrier()
    base = wid * EPW

    def eoff(i):
        return pl.multiple_of(base + i * CH, 8)

    def iload(i, p):
        pltpu.async_copy(dst_hbm.at[pl.ds(eoff(i), CH)], idx[p], isem[p])

    def iwait(i, p):
        pltpu.make_async_copy(dst_hbm.at[pl.ds(eoff(i), CH)],
                              idx[p], isem[p]).wait()

    def scatter(i, p):
        pltpu.async_copy(ones_v, acc.at[idx[p]], ssem[p], add=True)

    def swait(i, p):
        pltpu.make_async_copy(ones_v, acc.at[idx[p]], ssem[p]).wait()

    for k in range(4):
        iload(k, k)

    def step(i, p):
        q = (p + 4) % _NHB

        @pl.when(i >= 4)
        def _():
            swait(i - 4, q)

        @pl.when(i + 4 < NCHUNK)
        def _():
            iload(i + 4, q)

        iwait(i, p)
        scatter(i, p)

    def body(j, carry):
        for p in range(_NHB):
            step(_NHB * j + p, p)
        return carry

    lax.fori_loop(0, NCHUNK // _NHB, body, 0)
    for k in range(4):
        i = NCHUNK - 4 + k
        swait(i, i % _NHB)
    plsc.subcore_barrier()
    pltpu.sync_copy(acc.at[pl.ds(r0, RPT)],
                    deg_hbm.at[pl.ds(c * N2 + r0, RPT)])


@functools.partial(
    pl.kernel,
    out_type=jax.ShapeDtypeStruct((NC * N2, D), jnp.float32),
    mesh=_MESH,
    scratch_types=[
        pltpu.VMEM((CH,), jnp.int32),
        pltpu.VMEM((CH,), jnp.int32),
        pltpu.VMEM((CH,), jnp.int32),
        pltpu.VMEM((CH,), jnp.int32),
        pltpu.VMEM((CH, D), jnp.float32),
        pltpu.VMEM((CH, D), jnp.float32),
        pltpu.VMEM_SHARED((N2, D), jnp.float32),
    ] + [pltpu.SemaphoreType.DMA] * 6,
)
def _sc_msg(g_hbm, srcf_hbm, dstf_hbm, zeros_hbm, q_hbm,
            idxs0, idxs1, idxd0, idxd1, rows0, rows1, acc,
            gsem0, gsem1, isem0, isem1, dsem0, dsem1):
    c = lax.axis_index("c")
    s = lax.axis_index("s")
    wid = c * NS + s
    r0 = s * RPT
    pltpu.sync_copy(zeros_hbm.at[pl.ds(r0, RPT)], acc.at[pl.ds(r0, RPT)])
    plsc.subcore_barrier()

    rows = (rows0, rows1)
    idxs = (idxs0, idxs1)
    idxd = (idxd0, idxd1)
    gsem = (gsem0, gsem1)
    isem = (isem0, isem1)
    dsem = (dsem0, dsem1)
    base = wid * EPW

    def eoff(i):
        return pl.multiple_of(base + i * CH, 8)

    pltpu.sync_copy(srcf_hbm.at[pl.ds(eoff(0), CH)], idxs0)
    pltpu.async_copy(dstf_hbm.at[pl.ds(eoff(0), CH)], idxd0, dsem0)
    pltpu.async_copy(g_hbm.at[idxs0], rows0, gsem0)
    pltpu.async_copy(srcf_hbm.at[pl.ds(eoff(1), CH)], idxs1, isem1)

    def step(i, p):
        pltpu.make_async_copy(g_hbm.at[idxs[p]], rows[p], gsem[p]).wait()

        @pl.when(i + 2 < NCHUNK)
        def _():
            pltpu.async_copy(srcf_hbm.at[pl.ds(eoff(i + 2), CH)],
                             idxs[p], isem[p])

        @pl.when(i + 1 < NCHUNK)
        def _():
            pltpu.make_async_copy(srcf_hbm.at[pl.ds(eoff(i + 1), CH)],
                                  idxs[1 - p], isem[1 - p]).wait()
            pltpu.async_copy(g_hbm.at[idxs[1 - p]], rows[1 - p], gsem[1 - p])
            pltpu.async_copy(dstf_hbm.at[pl.ds(eoff(i + 1), CH)],
                             idxd[1 - p], dsem[1 - p])

        pltpu.make_async_copy(dstf_hbm.at[pl.ds(eoff(i), CH)],
                              idxd[p], dsem[p]).wait()
        pltpu.sync_copy(rows[p], acc.at[idxd[p]], add=True)

    def body(j, carry):
        step(2 * j, 0)
        step(2 * j + 1, 1)
        return carry

    lax.fori_loop(0, NCHUNK // 2, body, 0)
    plsc.subcore_barrier()
    pltpu.sync_copy(acc.at[pl.ds(r0, RPT)],
                    q_hbm.at[pl.ds(c * N2 + r0, RPT)])


def _tc_matmul_body(x_ref, w_ref, h_ref):
    h_ref[...] = jnp.dot(x_ref[...], w_ref[...],
                         preferred_element_type=jnp.float32)


def _tc_matmul(x2, W):
    return pl.pallas_call(
        _tc_matmul_body,
        grid=(NBLK,),
        in_specs=[
            pl.BlockSpec((BLK, D), lambda i: (i, 0)),
            pl.BlockSpec((D, D), lambda i: (0, 0)),
        ],
        out_specs=pl.BlockSpec((BLK, D), lambda i: (i, 0)),
        out_shape=jax.ShapeDtypeStruct((N2, D), jnp.float32),
    )(x2, W)


def _tc_scale_body(h_ref, degp_ref, g_ref, dinv_ref):
    deg = 1.0 + degp_ref[0] + degp_ref[1]
    di = lax.rsqrt(deg)
    g_ref[...] = h_ref[...] * di
    dinv_ref[...] = di


def _tc_scale(h, degp):
    return pl.pallas_call(
        _tc_scale_body,
        grid=(NBLK,),
        in_specs=[
            pl.BlockSpec((BLK, D), lambda i: (i, 0)),
            pl.BlockSpec((NC, BLK, 1), lambda i: (0, i, 0)),
        ],
        out_specs=[
            pl.BlockSpec((BLK, D), lambda i: (i, 0)),
            pl.BlockSpec((BLK, 1), lambda i: (i, 0)),
        ],
        out_shape=[
            jax.ShapeDtypeStruct((N2, D), jnp.float32),
            jax.ShapeDtypeStruct((N2, 1), jnp.float32),
        ],
    )(h, degp)


def _tc_combine_body(q0_ref, q1_ref, g_ref, dinv_ref, b_ref, o_ref):
    acc = q0_ref[...] + q1_ref[...] + g_ref[...]
    o_ref[...] = jnp.maximum(acc * dinv_ref[...] + b_ref[...], 0.0)


def _tc_combine(qf, g, dinv, b2):
    return pl.pallas_call(
        _tc_combine_body,
        grid=(NBLK,),
        in_specs=[
            pl.BlockSpec((BLK, D), lambda i: (i, 0)),
            pl.BlockSpec((BLK, D), lambda i: (i + NBLK, 0)),
            pl.BlockSpec((BLK, D), lambda i: (i, 0)),
            pl.BlockSpec((BLK, 1), lambda i: (i, 0)),
            pl.BlockSpec((1, D), lambda i: (0, 0)),
        ],
        out_specs=pl.BlockSpec((BLK, D), lambda i: (i, 0)),
        out_shape=jax.ShapeDtypeStruct((N2, D), jnp.float32),
    )(qf, qf, g, dinv, b2)


def kernel(x, edge_index, W, b):
    pad_idx = N + jnp.arange(E2 - E, dtype=jnp.int32) % (N2 - N)
    ei2 = jnp.concatenate(
        [edge_index, jnp.stack([pad_idx, pad_idx])], axis=1)
    src = ei2[0]
    dst = ei2[1]
    x2 = jnp.pad(x, ((0, N2 - N), (0, 0)))
    z1 = jnp.zeros((N2,), jnp.float32)
    zD = jnp.zeros((N2, D), jnp.float32)

    h = _tc_matmul(x2, W)
    degp = _sc_hist(dst, z1).reshape(NC, N2, 1)
    g, dinv = _tc_scale(h, degp)
    qf = _sc_msg(g, src, dst, zD)
    out = _tc_combine(qf, g, dinv, b.reshape(1, D))
    return out[:N]


# speedup vs baseline: 1.1613x; 1.0076x over previous
import functools

import jax
import jax.numpy as jnp
from jax import lax
from jax.experimental import pallas as pl
from jax.experimental.pallas import tpu as pltpu
from jax.experimental.pallas import tpu_sc as plsc

N, E, D = 10000, 320000, 128
E2 = 327680
N2 = 10240
NC, NS = 2, 16
NW = NC * NS
EPW = E2 // NW
CH = 128
NCHUNK = EPW // CH
RPT = N2 // NS
BLK = 1024
NBLK = N2 // BLK

_MESH = plsc.VectorSubcoreMesh(core_axis_name="c", subcore_axis_name="s")


_NHB = 8


@functools.partial(
    pl.kernel,
    out_type=jax.ShapeDtypeStruct((NC * N2,), jnp.float32),
    mesh=_MESH,
    scratch_types=[pltpu.VMEM((CH,), jnp.int32)] * _NHB
    + [
        pltpu.VMEM((CH,), jnp.float32),
        pltpu.VMEM_SHARED((N2,), jnp.float32),
    ]
    + [pltpu.SemaphoreType.DMA] * (2 * _NHB),
)
def _sc_hist(dst_hbm, zeros_hbm, deg_hbm, *refs):
    idx = refs[:_NHB]
    ones_v = refs[_NHB]
    acc = refs[_NHB + 1]
    isem = refs[_NHB + 2:_NHB + 2 + _NHB]
    ssem = refs[_NHB + 2 + _NHB:]
    c = lax.axis_index("c")
    s = lax.axis_index("s")
    wid = c * NS + s
    r0 = s * RPT
    pltpu.sync_copy(zeros_hbm.at[pl.ds(r0, RPT)], acc.at[pl.ds(r0, RPT)])
    for k in range(CH // 16):
        ones_v[pl.ds(k * 16, 16)] = jnp.full((16,), 1.0, jnp.float32)
    plsc.subcore_barrier()
    base = wid * EPW

    def eoff(i):
        return pl.multiple_of(base + i * CH, 8)

    def iload(i, p):
        pltpu.async_copy(dst_hbm.at[pl.ds(eoff(i), CH)], idx[p], isem[p])

    def iwait(i, p):
        pltpu.make_async_copy(dst_hbm.at[pl.ds(eoff(i), CH)],
                              idx[p], isem[p]).wait()

    def scatter(i, p):
        pltpu.async_copy(ones_v, acc.at[idx[p]], ssem[p], add=True)

    def swait(i, p):
        pltpu.make_async_copy(ones_v, acc.at[idx[p]], ssem[p]).wait()

    for k in range(4):
        iload(k, k)

    def step(i, p):
        q = (p + 4) % _NHB

        @pl.when(i >= 4)
        def _():
            swait(i - 4, q)

        @pl.when(i + 4 < NCHUNK)
        def _():
            iload(i + 4, q)

        iwait(i, p)
        scatter(i, p)

    def body(j, carry):
        for p in range(_NHB):
            step(_NHB * j + p, p)
        return carry

    lax.fori_loop(0, NCHUNK // _NHB, body, 0)
    for k in range(4):
        i = NCHUNK - 4 + k
        swait(i, i % _NHB)
    plsc.subcore_barrier()
    pltpu.sync_copy(acc.at[pl.ds(r0, RPT)],
                    deg_hbm.at[pl.ds(c * N2 + r0, RPT)])


@functools.partial(
    pl.kernel,
    out_type=jax.ShapeDtypeStruct((NC * N2, D), jnp.float32),
    mesh=_MESH,
    scratch_types=[
        pltpu.VMEM((CH,), jnp.int32),
        pltpu.VMEM((CH,), jnp.int32),
        pltpu.VMEM((CH,), jnp.int32),
        pltpu.VMEM((CH,), jnp.int32),
        pltpu.VMEM((CH, D), jnp.float32),
        pltpu.VMEM((CH, D), jnp.float32),
        pltpu.VMEM_SHARED((N2, D), jnp.float32),
    ] + [pltpu.SemaphoreType.DMA] * 6,
)
def _sc_msg(g_hbm, srcf_hbm, dstf_hbm, zeros_hbm, q_hbm,
            idxs0, idxs1, idxd0, idxd1, rows0, rows1, acc,
            gsem0, gsem1, isem0, isem1, dsem0, dsem1):
    c = lax.axis_index("c")
    s = lax.axis_index("s")
    wid = c * NS + s
    r0 = s * RPT
    pltpu.sync_copy(zeros_hbm.at[pl.ds(r0, RPT)], acc.at[pl.ds(r0, RPT)])
    plsc.subcore_barrier()

    rows = (rows0, rows1)
    idxs = (idxs0, idxs1)
    idxd = (idxd0, idxd1)
    gsem = (gsem0, gsem1)
    isem = (isem0, isem1)
    dsem = (dsem0, dsem1)
    base = wid * EPW

    def eoff(i):
        return pl.multiple_of(base + i * CH, 8)

    pltpu.sync_copy(srcf_hbm.at[pl.ds(eoff(0), CH)], idxs0)
    pltpu.async_copy(dstf_hbm.at[pl.ds(eoff(0), CH)], idxd0, dsem0)
    pltpu.async_copy(g_hbm.at[idxs0], rows0, gsem0)
    pltpu.async_copy(srcf_hbm.at[pl.ds(eoff(1), CH)], idxs1, isem1)

    def step(i, p):
        pltpu.make_async_copy(g_hbm.at[idxs[p]], rows[p], gsem[p]).wait()

        @pl.when(i + 2 < NCHUNK)
        def _():
            pltpu.async_copy(srcf_hbm.at[pl.ds(eoff(i + 2), CH)],
                             idxs[p], isem[p])

        @pl.when(i + 1 < NCHUNK)
        def _():
            pltpu.make_async_copy(srcf_hbm.at[pl.ds(eoff(i + 1), CH)],
                                  idxs[1 - p], isem[1 - p]).wait()
            pltpu.async_copy(g_hbm.at[idxs[1 - p]], rows[1 - p], gsem[1 - p])
            pltpu.async_copy(dstf_hbm.at[pl.ds(eoff(i + 1), CH)],
                             idxd[1 - p], dsem[1 - p])

        pltpu.make_async_copy(dstf_hbm.at[pl.ds(eoff(i), CH)],
                              idxd[p], dsem[p]).wait()
        pltpu.sync_copy(rows[p], acc.at[idxd[p]], add=True)

    def body(j, carry):
        step(2 * j, 0)
        step(2 * j + 1, 1)
        return carry

    lax.fori_loop(0, NCHUNK // 2, body, 0)
    plsc.subcore_barrier()
    pltpu.sync_copy(acc.at[pl.ds(r0, RPT)],
                    q_hbm.at[pl.ds(c * N2 + r0, RPT)])


def _tc_scale_body(x_ref, w_ref, degp_ref, g_ref, dinv_ref):
    deg = 1.0 + degp_ref[0] + degp_ref[1]
    di = lax.rsqrt(deg)
    h = jnp.dot(x_ref[...], w_ref[...], preferred_element_type=jnp.float32)
    g_ref[...] = h * di
    dinv_ref[...] = di


def _tc_scale(x2, W, degp):
    return pl.pallas_call(
        _tc_scale_body,
        grid=(NBLK,),
        in_specs=[
            pl.BlockSpec((BLK, D), lambda i: (i, 0)),
            pl.BlockSpec((D, D), lambda i: (0, 0)),
            pl.BlockSpec((NC, BLK, 1), lambda i: (0, i, 0)),
        ],
        out_specs=[
            pl.BlockSpec((BLK, D), lambda i: (i, 0)),
            pl.BlockSpec((BLK, 1), lambda i: (i, 0)),
        ],
        out_shape=[
            jax.ShapeDtypeStruct((N2, D), jnp.float32),
            jax.ShapeDtypeStruct((N2, 1), jnp.float32),
        ],
    )(x2, W, degp)


def _tc_combine_body(q0_ref, q1_ref, g_ref, dinv_ref, b_ref, o_ref):
    acc = q0_ref[...] + q1_ref[...] + g_ref[...]
    o_ref[...] = jnp.maximum(acc * dinv_ref[...] + b_ref[...], 0.0)


def _tc_combine(qf, g, dinv, b2):
    return pl.pallas_call(
        _tc_combine_body,
        grid=(NBLK,),
        in_specs=[
            pl.BlockSpec((BLK, D), lambda i: (i, 0)),
            pl.BlockSpec((BLK, D), lambda i: (i + NBLK, 0)),
            pl.BlockSpec((BLK, D), lambda i: (i, 0)),
            pl.BlockSpec((BLK, 1), lambda i: (i, 0)),
            pl.BlockSpec((1, D), lambda i: (0, 0)),
        ],
        out_specs=pl.BlockSpec((BLK, D), lambda i: (i, 0)),
        out_shape=jax.ShapeDtypeStruct((N2, D), jnp.float32),
    )(qf, qf, g, dinv, b2)


def kernel(x, edge_index, W, b):
    pad_idx = N + jnp.arange(E2 - E, dtype=jnp.int32) % (N2 - N)
    ei2 = jnp.concatenate(
        [edge_index, jnp.stack([pad_idx, pad_idx])], axis=1)
    src = ei2[0]
    dst = ei2[1]
    x2 = jnp.pad(x, ((0, N2 - N), (0, 0)))
    z1 = jnp.zeros((N2,), jnp.float32)
    zD = jnp.zeros((N2, D), jnp.float32)

    degp = _sc_hist(dst, z1).reshape(NC, N2, 1)
    g, dinv = _tc_scale(x2, W, degp)
    qf = _sc_msg(g, src, dst, zD)
    out = _tc_combine(qf, g, dinv, b.reshape(1, D))
    return out[:N]
